# Initial kernel scaffold; baseline (speedup 1.0000x reference)
import jax
import jax.numpy as jnp
from jax.experimental import pallas as pl


def kernel(pred, targets):
    raise NotImplementedError("write your pallas kernel here")


# speedup vs baseline: 2.5501x; 2.5501x over previous
import functools

import jax
import jax.numpy as jnp
from jax import lax
from jax.experimental import pallas as pl
from jax.experimental.pallas import tpu as pltpu
from jax.experimental.pallas import tpu_sc as plsc

_IMG = 320.0
_NCLS = 30
_POSW = 20.0
_LEVELS = ((8, 40, 0), (16, 20, 1600), (32, 10, 2000))
_B, _T, _N, _C = 8, 50, 2100, 35
_TPAD = 64


def _sc_assign_body(tx_hbm, ty_hbm, idx_hbm, tx_v, ty_v, idx_v):
    wid = lax.axis_index("s") * 2 + lax.axis_index("c")
    base = wid * 16
    pltpu.sync_copy(tx_hbm.at[pl.ds(base, 16)], tx_v)
    pltpu.sync_copy(ty_hbm.at[pl.ds(base, 16)], ty_v)
    x = tx_v[...] * _IMG
    y = ty_v[...] * _IMG
    best_d = jnp.full((16,), 1e30, jnp.float32)
    best_i = jnp.zeros((16,), jnp.int32)
    for s, g, off in _LEVELS:
        inv = 1.0 / s
        gx0 = (x * inv).astype(jnp.int32)
        gy0 = (y * inv).astype(jnp.int32)
        for dgy in (-1, 0, 1):
            for dgx in (-1, 0, 1):
                gx = jnp.clip(gx0 + dgx, 0, g - 1)
                gy = jnp.clip(gy0 + dgy, 0, g - 1)
                ax = (gx.astype(jnp.float32) + 0.5) * s
                ay = (gy.astype(jnp.float32) + 0.5) * s
                dx = x - ax
                dy = y - ay
                d = dx * dx + dy * dy
                i = off + gy * g + gx
                better = (d < best_d) | ((d == best_d) & (i < best_i))
                best_d = jnp.where(better, d, best_d)
                best_i = jnp.where(better, i, best_i)
    idx_v[...] = best_i
    pltpu.sync_copy(idx_v, idx_hbm.at[pl.ds(base, 16)])


@functools.cache
def _sc_assign():
    return pl.kernel(
        _sc_assign_body,
        out_type=jax.ShapeDtypeStruct((_B * _TPAD,), jnp.int32),
        mesh=plsc.VectorSubcoreMesh(core_axis_name="c", subcore_axis_name="s"),
        scratch_types=[
            pltpu.VMEM((16,), jnp.float32),
            pltpu.VMEM((16,), jnp.float32),
            pltpu.VMEM((16,), jnp.int32),
        ],
    )


def _softplus(x):
    return jnp.maximum(x, 0.0) + jnp.log1p(jnp.exp(-jnp.abs(x)))


def _tc_loss_body(pred_ref, tgt_ref, idx_ref, out_ref):
    b = pl.program_id(0)
    pred_b = pred_ref[0]
    tgt = tgt_ref[0]
    idxc = idx_ref[0]

    io_n = lax.broadcasted_iota(jnp.int32, (_T, _N), 1)
    onehot = (io_n == idxc).astype(jnp.float32)

    g = lax.dot_general(
        onehot, pred_b, (((1,), (1,)), ((), ())),
        preferred_element_type=jnp.float32,
    )

    boxes = jax.nn.sigmoid(g[:, 0:4])
    dbox = boxes - tgt[:, 1:5]
    loss_box = jnp.sum(dbox * dbox) * (1.0 / (_T * 4))

    cls_logits = g[:, 5:_C]
    cls_idx = tgt[:, 0:1].astype(jnp.int32)
    io_c = lax.broadcasted_iota(jnp.int32, (_T, _NCLS), 1)
    z = (io_c == cls_idx).astype(jnp.float32)
    loss_cls = jnp.sum(
        z * _softplus(-cls_logits) + (1.0 - z) * _softplus(cls_logits)
    ) * (1.0 / (_T * _NCLS))

    tobj = jnp.max(onehot, axis=0, keepdims=True)
    po = pred_b[4:5, :]
    loss_obj = jnp.sum(
        _POSW * tobj * _softplus(-po) + (1.0 - tobj) * _softplus(po)
    ) * (1.0 / _N)

    total = 5.0 * loss_box + loss_obj + loss_cls

    @pl.when(b == 0)
    def _():
        out_ref[...] = jnp.zeros((1, 1), jnp.float32)

    out_ref[...] = out_ref[...] + total


def _tc_loss(pred, targets, idx_col):
    out = pl.pallas_call(
        _tc_loss_body,
        grid=(_B,),
        in_specs=[
            pl.BlockSpec((1, _C, _N), lambda b: (b, 0, 0)),
            pl.BlockSpec((1, _T, 5), lambda b: (b, 0, 0)),
            pl.BlockSpec((1, _T, 1), lambda b: (b, 0, 0)),
        ],
        out_specs=pl.BlockSpec((1, 1), lambda b: (0, 0)),
        out_shape=jax.ShapeDtypeStruct((1, 1), jnp.float32),
        compiler_params=pltpu.CompilerParams(
            dimension_semantics=("arbitrary",)
        ),
    )(pred, targets, idx_col)
    return out[0, 0]


def kernel(pred, targets):
    t_xy = targets[:, :, 1:3]
    pad = jnp.zeros((_B, _TPAD - _T), jnp.float32)
    tx = jnp.concatenate([t_xy[:, :, 0], pad], axis=1).reshape(-1)
    ty = jnp.concatenate([t_xy[:, :, 1], pad], axis=1).reshape(-1)
    idx_flat = _sc_assign()(tx, ty)
    idx_col = idx_flat.reshape(_B, _TPAD)[:, :_T][..., None]
    return _tc_loss(pred, targets, idx_col)


# speedup vs baseline: 2.8480x; 1.1168x over previous
import functools

import jax
import jax.numpy as jnp
from jax import lax
from jax.experimental import pallas as pl
from jax.experimental.pallas import tpu as pltpu
from jax.experimental.pallas import tpu_sc as plsc

_IMG = 320.0
_NCLS = 30
_POSW = 20.0
_LEVELS = ((8, 40, 0), (16, 20, 1600), (32, 10, 2000))
_B, _T, _N, _C = 8, 50, 2100, 35
_TPAD = 64


def _sc_assign_body(xy_hbm, idx_hbm, x_v, y_v, idx_v):
    wid = lax.axis_index("s") * 2 + lax.axis_index("c")
    base = wid * 16
    base_t = (wid % 4) * 16
    pltpu.sync_copy(xy_hbm.at[0, pl.ds(base, 16)], x_v)
    pltpu.sync_copy(xy_hbm.at[1, pl.ds(base, 16)], y_v)
    lane = jax.lax.iota(jnp.int32, 16)
    valid = base_t + lane < _T
    x = x_v[...] * _IMG
    y = y_v[...] * _IMG
    best_d = jnp.full((16,), 1e30, jnp.float32)
    best_i = jnp.zeros((16,), jnp.int32)
    for s, g, off in _LEVELS:
        inv = 1.0 / s
        gx0 = (x * inv).astype(jnp.int32)
        gy0 = (y * inv).astype(jnp.int32)
        for dgy in (-1, 0, 1):
            for dgx in (-1, 0, 1):
                gx = jnp.clip(gx0 + dgx, 0, g - 1)
                gy = jnp.clip(gy0 + dgy, 0, g - 1)
                ax = (gx.astype(jnp.float32) + 0.5) * s
                ay = (gy.astype(jnp.float32) + 0.5) * s
                dx = x - ax
                dy = y - ay
                d = dx * dx + dy * dy
                i = off + gy * g + gx
                better = (d < best_d) | ((d == best_d) & (i < best_i))
                best_d = jnp.where(better, d, best_d)
                best_i = jnp.where(better, i, best_i)
    idx_v[...] = jnp.where(valid, best_i, -1)
    pltpu.sync_copy(idx_v, idx_hbm.at[pl.ds(base, 16)])


@functools.cache
def _sc_assign():
    return pl.kernel(
        _sc_assign_body,
        out_type=jax.ShapeDtypeStruct((_B * _TPAD,), jnp.int32),
        mesh=plsc.VectorSubcoreMesh(core_axis_name="c", subcore_axis_name="s"),
        scratch_types=[
            pltpu.VMEM((16,), jnp.float32),
            pltpu.VMEM((16,), jnp.float32),
            pltpu.VMEM((16,), jnp.int32),
        ],
    )


def _softplus(x):
    return jnp.maximum(x, 0.0) + jnp.log1p(jnp.exp(-jnp.abs(x)))


def _tc_loss_body(pred_ref, tgt_ref, idx_ref, out_ref):
    total = jnp.zeros((1, 1), jnp.float32)
    for b in range(_B):
        pred_b = pred_ref[b]
        tgt = tgt_ref[b]
        idxc = idx_ref[b]

        io_n = lax.broadcasted_iota(jnp.int32, (_TPAD, _N), 1)
        onehot = (io_n == idxc).astype(jnp.float32)

        g = lax.dot_general(
            onehot, pred_b, (((1,), (1,)), ((), ())),
            preferred_element_type=jnp.float32,
        )

        boxes = jax.nn.sigmoid(g[:_T, 0:4])
        dbox = boxes - tgt[:, 1:5]
        loss_box = jnp.sum(dbox * dbox) * (1.0 / (_T * 4))

        cls_logits = g[:_T, 5:_C]
        cls_idx = tgt[:, 0:1].astype(jnp.int32)
        io_c = lax.broadcasted_iota(jnp.int32, (_T, _NCLS), 1)
        z = (io_c == cls_idx).astype(jnp.float32)
        loss_cls = jnp.sum(
            z * _softplus(-cls_logits) + (1.0 - z) * _softplus(cls_logits)
        ) * (1.0 / (_T * _NCLS))

        tobj = jnp.max(onehot, axis=0, keepdims=True)
        po = pred_b[4:5, :]
        loss_obj = jnp.sum(
            _POSW * tobj * _softplus(-po) + (1.0 - tobj) * _softplus(po)
        ) * (1.0 / _N)

        total = total + (5.0 * loss_box + loss_obj + loss_cls)
    out_ref[...] = total


def _tc_loss(pred, targets, idx_col):
    out = pl.pallas_call(
        _tc_loss_body,
        out_shape=jax.ShapeDtypeStruct((1, 1), jnp.float32),
    )(pred, targets, idx_col)
    return out[0, 0]


def kernel(pred, targets):
    t_xy = targets[:, :, 1:3]
    xy = jnp.moveaxis(
        jnp.pad(t_xy, ((0, 0), (0, _TPAD - _T), (0, 0))), 2, 0
    ).reshape(2, _B * _TPAD)
    idx_flat = _sc_assign()(xy)
    idx_col = idx_flat.reshape(_B, _TPAD, 1)
    return _tc_loss(pred, targets, idx_col)


# speedup vs baseline: 3.0938x; 1.0863x over previous
import functools

import jax
import jax.numpy as jnp
from jax import lax
from jax.experimental import pallas as pl
from jax.experimental.pallas import tpu as pltpu
from jax.experimental.pallas import tpu_sc as plsc

_IMG = 320.0
_NCLS = 30
_POSW = 20.0
_LEVELS = ((8, 40, 0), (16, 20, 1600), (32, 10, 2000))
_B, _T, _N, _C = 8, 50, 2100, 35
_TPAD = 64


def _sc_assign_body(xy_hbm, idx_hbm, x_v, y_v, idx_v):
    wid = lax.axis_index("s") * 2 + lax.axis_index("c")
    base = wid * 16
    base_t = (wid % 4) * 16
    pltpu.sync_copy(xy_hbm.at[0, pl.ds(base, 16)], x_v)
    pltpu.sync_copy(xy_hbm.at[1, pl.ds(base, 16)], y_v)
    lane = jax.lax.iota(jnp.int32, 16)
    valid = base_t + lane < _T
    x = x_v[...] * _IMG
    y = y_v[...] * _IMG
    best_d = jnp.full((16,), 1e30, jnp.float32)
    best_i = jnp.zeros((16,), jnp.int32)
    for s, g, off in _LEVELS:
        inv = 1.0 / s
        gx0 = (x * inv).astype(jnp.int32)
        gy0 = (y * inv).astype(jnp.int32)
        for dgy in (-1, 0, 1):
            for dgx in (-1, 0, 1):
                gx = jnp.clip(gx0 + dgx, 0, g - 1)
                gy = jnp.clip(gy0 + dgy, 0, g - 1)
                ax = (gx.astype(jnp.float32) + 0.5) * s
                ay = (gy.astype(jnp.float32) + 0.5) * s
                dx = x - ax
                dy = y - ay
                d = dx * dx + dy * dy
                i = off + gy * g + gx
                better = (d < best_d) | ((d == best_d) & (i < best_i))
                best_d = jnp.where(better, d, best_d)
                best_i = jnp.where(better, i, best_i)
    idx_v[...] = jnp.where(valid, best_i, -1)
    pltpu.sync_copy(idx_v, idx_hbm.at[pl.ds(base, 16)])


@functools.cache
def _sc_assign():
    return pl.kernel(
        _sc_assign_body,
        out_type=jax.ShapeDtypeStruct((_B * _TPAD,), jnp.int32),
        mesh=plsc.VectorSubcoreMesh(core_axis_name="c", subcore_axis_name="s"),
        scratch_types=[
            pltpu.VMEM((16,), jnp.float32),
            pltpu.VMEM((16,), jnp.float32),
            pltpu.VMEM((16,), jnp.int32),
        ],
    )


def _softplus(x):
    return jnp.maximum(x, 0.0) + jnp.log1p(jnp.exp(-jnp.abs(x)))


def _tc_loss_body(pred_ref, tgt_ref, idx_ref, out_ref):
    idx_all = idx_ref[...]
    io_r = lax.broadcasted_iota(jnp.int32, (_TPAD, _TPAD), 0)
    io_c = lax.broadcasted_iota(jnp.int32, (_TPAD, _TPAD), 1)
    eye = (io_r == io_c).astype(jnp.int32)
    total = jnp.zeros((1, 1), jnp.float32)
    for b in range(_B):
        pred_b = pred_ref[b]
        tgt = tgt_ref[b]
        row = idx_all[b * _TPAD:(b + 1) * _TPAD].reshape(1, _TPAD)
        idxc = jnp.sum(
            jnp.broadcast_to(row, (_TPAD, _TPAD)) * eye, axis=1, keepdims=True
        )

        io_n = lax.broadcasted_iota(jnp.int32, (_TPAD, _N), 1)
        onehot = (io_n == idxc).astype(jnp.float32)

        g = lax.dot_general(
            onehot, pred_b, (((1,), (1,)), ((), ())),
            preferred_element_type=jnp.float32,
        )

        boxes = jax.nn.sigmoid(g[:_T, 0:4])
        dbox = boxes - tgt[:, 1:5]
        loss_box = jnp.sum(dbox * dbox) * (1.0 / (_T * 4))

        cls_logits = g[:_T, 5:_C]
        cls_idx = tgt[:, 0:1].astype(jnp.int32)
        io_cl = lax.broadcasted_iota(jnp.int32, (_T, _NCLS), 1)
        z = (io_cl == cls_idx).astype(jnp.float32)
        loss_cls = jnp.sum(_softplus(cls_logits) - z * cls_logits) * (
            1.0 / (_T * _NCLS)
        )

        tobj = jnp.max(onehot, axis=0, keepdims=True)
        po = pred_b[4:5, :]
        loss_obj = jnp.sum(
            _softplus(po) * (1.0 + (_POSW - 1.0) * tobj) - _POSW * tobj * po
        ) * (1.0 / _N)

        total = total + (5.0 * loss_box + loss_obj + loss_cls)
    out_ref[...] = total


def _tc_loss(pred, targets, idx_flat):
    out = pl.pallas_call(
        _tc_loss_body,
        out_shape=jax.ShapeDtypeStruct((1, 1), jnp.float32),
    )(pred, targets, idx_flat)
    return out[0, 0]


def kernel(pred, targets):
    t_xy = targets[:, :, 1:3]
    xy = jnp.moveaxis(
        jnp.pad(t_xy, ((0, 0), (0, _TPAD - _T), (0, 0))), 2, 0
    ).reshape(2, _B * _TPAD)
    idx_flat = _sc_assign()(xy)
    return _tc_loss(pred, targets, idx_flat)


# speedup vs baseline: 3.1733x; 1.0257x over previous
import functools

import jax
import jax.numpy as jnp
from jax import lax
from jax.experimental import pallas as pl
from jax.experimental.pallas import tpu as pltpu
from jax.experimental.pallas import tpu_sc as plsc

_IMG = 320.0
_NCLS = 30
_POSW = 20.0
_LEVELS = ((8, 40, 0), (16, 20, 1600), (32, 10, 2000))
_B, _T, _N, _C = 8, 50, 2100, 35
_TPAD = 64


def _sc_assign_body(xy_hbm, idx_hbm, x_v, y_v, idx_v):
    sid = lax.axis_index("s")
    for c in range(2):
        wid = sid * 2 + c
        base = wid * 16
        base_t = (wid % 4) * 16
        pltpu.sync_copy(xy_hbm.at[0, pl.ds(base, 16)], x_v)
        pltpu.sync_copy(xy_hbm.at[1, pl.ds(base, 16)], y_v)
        lane = jax.lax.iota(jnp.int32, 16)
        valid = base_t + lane < _T
        x = x_v[...] * _IMG
        y = y_v[...] * _IMG
        best_d = jnp.full((16,), 1e30, jnp.float32)
        best_i = jnp.zeros((16,), jnp.int32)
        for s, g, off in _LEVELS:
            inv = 1.0 / s
            gx0 = (x * inv).astype(jnp.int32)
            gy0 = (y * inv).astype(jnp.int32)
            for dgy in (-1, 0, 1):
                for dgx in (-1, 0, 1):
                    gx = jnp.clip(gx0 + dgx, 0, g - 1)
                    gy = jnp.clip(gy0 + dgy, 0, g - 1)
                    ax = (gx.astype(jnp.float32) + 0.5) * s
                    ay = (gy.astype(jnp.float32) + 0.5) * s
                    dx = x - ax
                    dy = y - ay
                    d = dx * dx + dy * dy
                    i = off + gy * g + gx
                    better = (d < best_d) | ((d == best_d) & (i < best_i))
                    best_d = jnp.where(better, d, best_d)
                    best_i = jnp.where(better, i, best_i)
        idx_v[...] = jnp.where(valid, best_i, -1)
        pltpu.sync_copy(idx_v, idx_hbm.at[pl.ds(base, 16)])


@functools.cache
def _sc_assign():
    return pl.kernel(
        _sc_assign_body,
        out_type=jax.ShapeDtypeStruct((_B * _TPAD,), jnp.int32),
        mesh=plsc.VectorSubcoreMesh(
            core_axis_name="c", subcore_axis_name="s", num_cores=1
        ),
        scratch_types=[
            pltpu.VMEM((16,), jnp.float32),
            pltpu.VMEM((16,), jnp.float32),
            pltpu.VMEM((16,), jnp.int32),
        ],
    )


def _softplus(x):
    return jnp.maximum(x, 0.0) + jnp.log1p(jnp.exp(-jnp.abs(x)))


def _tc_loss_body(pred_ref, tgt_ref, idx_ref, out_ref):
    idx_all = idx_ref[...]
    io_r = lax.broadcasted_iota(jnp.int32, (_TPAD, _TPAD), 0)
    io_c = lax.broadcasted_iota(jnp.int32, (_TPAD, _TPAD), 1)
    eye = (io_r == io_c).astype(jnp.int32)
    total = jnp.zeros((1, 1), jnp.float32)
    for b in range(_B):
        pred_b = pred_ref[b]
        tgt = tgt_ref[b]
        row = idx_all[b * _TPAD:(b + 1) * _TPAD].reshape(1, _TPAD)
        idxc = jnp.sum(
            jnp.broadcast_to(row, (_TPAD, _TPAD)) * eye, axis=1, keepdims=True
        )

        io_n = lax.broadcasted_iota(jnp.int32, (_TPAD, _N), 1)
        onehot = (io_n == idxc).astype(jnp.float32)

        g = lax.dot_general(
            onehot, pred_b, (((1,), (1,)), ((), ())),
            preferred_element_type=jnp.float32,
        )

        boxes = jax.nn.sigmoid(g[:_T, 0:4])
        dbox = boxes - tgt[:, 1:5]
        loss_box = jnp.sum(dbox * dbox) * (1.0 / (_T * 4))

        cls_logits = g[:_T, 5:_C]
        cls_idx = tgt[:, 0:1].astype(jnp.int32)
        io_cl = lax.broadcasted_iota(jnp.int32, (_T, _NCLS), 1)
        z = (io_cl == cls_idx).astype(jnp.float32)
        loss_cls = jnp.sum(_softplus(cls_logits) - z * cls_logits) * (
            1.0 / (_T * _NCLS)
        )

        tobj = jnp.max(onehot, axis=0, keepdims=True)
        po = pred_b[4:5, :]
        loss_obj = jnp.sum(
            _softplus(po) * (1.0 + (_POSW - 1.0) * tobj) - _POSW * tobj * po
        ) * (1.0 / _N)

        total = total + (5.0 * loss_box + loss_obj + loss_cls)
    out_ref[...] = total


def _tc_loss(pred, targets, idx_flat):
    out = pl.pallas_call(
        _tc_loss_body,
        out_shape=jax.ShapeDtypeStruct((1, 1), jnp.float32),
    )(pred, targets, idx_flat)
    return out[0, 0]


def kernel(pred, targets):
    t_xy = targets[:, :, 1:3]
    xy = jnp.moveaxis(
        jnp.pad(t_xy, ((0, 0), (0, _TPAD - _T), (0, 0))), 2, 0
    ).reshape(2, _B * _TPAD)
    idx_flat = _sc_assign()(xy)
    return _tc_loss(pred, targets, idx_flat)
